# baseline (speedup 1.0000x reference)
import functools

import jax
import jax.numpy as jnp
from jax import lax
from jax.experimental import pallas as pl
from jax.experimental.pallas import tpu as pltpu
from jax.experimental.pallas import tpu_sc as plsc

_B = 16384
_D = 64
_N = 1000000
_NC = 2
_NS = 16
_NW = _NC * _NS
_BPW = _B // _NW
_L = 16
_G = _BPW // _L
_W = 128
_R = 4
_RB = 8


@functools.partial(
    pl.kernel,
    mesh=plsc.VectorSubcoreMesh(core_axis_name="c", subcore_axis_name="s"),
    out_type=jax.ShapeDtypeStruct((_B,), jnp.float32),
    compiler_params=pltpu.CompilerParams(
        needs_layout_passes=False, use_tc_tiling_on_sc=True),
    scratch_types=[
        pltpu.VMEM((_BPW,), jnp.int32),
        pltpu.VMEM((_BPW,), jnp.int32),
    ] + [pltpu.VMEM((_D, _W), jnp.float32)] * (2 * _R)
      + [pltpu.VMEM((1, _W), jnp.float32)] * (2 * _RB)
      + [
        pltpu.VMEM((_L,), jnp.float32),
        pltpu.VMEM((_BPW,), jnp.float32),
    ] + [pltpu.SemaphoreType.DMA] * (_R + _RB),
)
def _mf_score(uid_hbm, iid_hbm, uT_hbm, iT_hbm, ubT_hbm, ibT_hbm,
              gb_hbm, out_hbm,
              uid_v, iid_v, *rest):
    ue = rest[0:_R]
    ie = rest[_R:2 * _R]
    ub = rest[2 * _R:2 * _R + _RB]
    ib = rest[2 * _R + _RB:2 * _R + 2 * _RB]
    gb_v, out_v = rest[2 * _R + 2 * _RB], rest[2 * _R + 2 * _RB + 1]
    _sems = rest[2 * _R + 2 * _RB + 2:]
    sems, bsems = _sems[:_R], _sems[_R:]

    wid = lax.axis_index("s") * _NC + lax.axis_index("c")
    base = wid * _BPW

    pltpu.sync_copy(uid_hbm.at[pl.ds(base, _BPW)], uid_v)
    pltpu.sync_copy(iid_hbm.at[pl.ds(base, _BPW)], iid_v)
    pltpu.sync_copy(gb_hbm, gb_v)

    def fire(uscalar, iscalar, p):
        uoff = pl.multiple_of((uscalar >> 7) * _W, _W)
        ioff = pl.multiple_of((iscalar >> 7) * _W, _W)
        pltpu.async_copy(uT_hbm.at[:, pl.ds(uoff, _W)], ue[p], sems[p])
        pltpu.async_copy(iT_hbm.at[:, pl.ds(ioff, _W)], ie[p], sems[p])

    def bfire(uscalar, iscalar, p):
        uoff = pl.multiple_of((uscalar >> 7) * _W, _W)
        ioff = pl.multiple_of((iscalar >> 7) * _W, _W)
        pltpu.async_copy(ubT_hbm.at[:, pl.ds(uoff, _W)], ub[p], bsems[p])
        pltpu.async_copy(ibT_hbm.at[:, pl.ds(ioff, _W)], ib[p], bsems[p])

    def drain(p):
        pltpu.make_async_copy(uT_hbm.at[:, pl.ds(0, _W)], ue[p], sems[p]).wait()
        pltpu.make_async_copy(iT_hbm.at[:, pl.ds(0, _W)], ie[p], sems[p]).wait()

    def bdrain(p):
        pltpu.make_async_copy(
            ubT_hbm.at[:, pl.ds(0, _W)], ub[p], bsems[p]).wait()
        pltpu.make_async_copy(
            ibT_hbm.at[:, pl.ds(0, _W)], ib[p], bsems[p]).wait()

    u16p = uid_v[pl.ds(0, _L)]
    i16p = iid_v[pl.ds(0, _L)]
    for p in range(_R):
        fire(u16p[p], i16p[p], p)
    for p in range(_RB):
        bfire(u16p[p], i16p[p], p)

    gb0 = gb_v[pl.ds(0, _L)][0]
    lane = lax.iota(jnp.int32, _L)
    zero16 = jnp.zeros((_L,), jnp.int32)

    def group(g, acc16):
        u16 = uid_v[pl.ds(g * _L, _L)]
        i16 = iid_v[pl.ds(g * _L, _L)]
        gnext = jnp.minimum(g + 1, _G - 1) * _L
        un16 = uid_v[pl.ds(gnext, _L)]
        in16 = iid_v[pl.ds(gnext, _L)]

        for j in range(_L):
            p = j % _R
            pb = j % _RB
            drain(p)
            bdrain(pb)
            cu = jnp.full((_L,), u16[j] & (_W - 1), jnp.int32)
            ci = jnp.full((_L,), i16[j] & (_W - 1), jnp.int32)
            s = jnp.zeros((_L,), jnp.float32)
            for q in range(_D // _L):
                rows = q * _L + lane
                u = plsc.load_gather(ue[p], [rows, cu])
                it = plsc.load_gather(ie[p], [rows, ci])
                s = s + u * it
            bias = (plsc.load_gather(ub[pb], [zero16, cu])
                    + plsc.load_gather(ib[pb], [zero16, ci]))

            if j < _L - _R:
                fire(u16[j + _R], i16[j + _R], p)
            else:
                @pl.when(g < _G - 1)
                def _(j=j, p=p):
                    fire(un16[j + _R - _L], in16[j + _R - _L], p)
            if j < _L - _RB:
                bfire(u16[j + _RB], i16[j + _RB], pb)
            else:
                @pl.when(g < _G - 1)
                def _(j=j, pb=pb):
                    bfire(un16[j + _RB - _L], in16[j + _RB - _L], pb)

            total = jnp.sum(s) + bias[0] + gb0
            acc16 = jnp.where(lane == j, total, acc16)

        out_v[pl.ds(g * _L, _L)] = acc16
        return acc16

    lax.fori_loop(0, _G, group, jnp.zeros((_L,), jnp.float32))

    pltpu.sync_copy(out_v, out_hbm.at[pl.ds(base, _BPW)])


def kernel(user_ids, item_ids, user_emb, item_emb, user_bias, item_bias,
           global_bias):
    uid = user_ids.astype(jnp.int32)
    iid = item_ids.astype(jnp.int32)
    return _mf_score(uid, iid, user_emb.T, item_emb.T, user_bias.T,
                     item_bias.T,
                     jnp.broadcast_to(global_bias.reshape(-1)[:1], (_L,)))


# speedup vs baseline: 1.0468x; 1.0468x over previous
import functools

import jax
import jax.numpy as jnp
from jax import lax
from jax.experimental import pallas as pl
from jax.experimental.pallas import tpu as pltpu
from jax.experimental.pallas import tpu_sc as plsc

_B = 16384
_D = 64
_N = 1000000
_NC = 2
_NS = 16
_NW = _NC * _NS
_BPW = _B // _NW
_L = 16
_G = _BPW // _L
_W = 128
_RU = 8
_R = 4
_RB = 8


@functools.partial(
    pl.kernel,
    mesh=plsc.VectorSubcoreMesh(core_axis_name="c", subcore_axis_name="s"),
    out_type=jax.ShapeDtypeStruct((_B,), jnp.float32),
    compiler_params=pltpu.CompilerParams(
        needs_layout_passes=False, use_tc_tiling_on_sc=True),
    scratch_types=[
        pltpu.VMEM((_BPW,), jnp.int32),
        pltpu.VMEM((_BPW,), jnp.int32),
    ] + [pltpu.VMEM((_D, _W), jnp.float32)] * (_RU + _R)
      + [pltpu.VMEM((1, _W), jnp.float32)] * (2 * _RB)
      + [
        pltpu.VMEM((_L,), jnp.float32),
        pltpu.VMEM((_BPW,), jnp.float32),
    ] + [pltpu.SemaphoreType.DMA] * (_RU + _R + _RB),
)
def _mf_score(uid_hbm, iid_hbm, uT_hbm, iT_hbm, ubT_hbm, ibT_hbm,
              gb_hbm, out_hbm,
              uid_v, iid_v, *rest):
    _ne = _RU + _R
    ue = rest[0:_RU]
    ie = rest[_RU:_ne]
    ub = rest[_ne:_ne + _RB]
    ib = rest[_ne + _RB:_ne + 2 * _RB]
    gb_v, out_v = rest[_ne + 2 * _RB], rest[_ne + 2 * _RB + 1]
    _sems = rest[_ne + 2 * _RB + 2:]
    usems, sems, bsems = _sems[:_RU], _sems[_RU:_ne], _sems[_ne:]

    wid = lax.axis_index("s") * _NC + lax.axis_index("c")
    base = wid * _BPW

    pltpu.sync_copy(uid_hbm.at[pl.ds(base, _BPW)], uid_v)
    pltpu.sync_copy(iid_hbm.at[pl.ds(base, _BPW)], iid_v)
    pltpu.sync_copy(gb_hbm, gb_v)

    def ufire(uscalar, pu):
        uoff = pl.multiple_of((uscalar >> 7) * _W, _W)
        pltpu.async_copy(uT_hbm.at[:, pl.ds(uoff, _W)], ue[pu], usems[pu])

    def fire(iscalar, p):
        ioff = pl.multiple_of((iscalar >> 7) * _W, _W)
        pltpu.async_copy(iT_hbm.at[:, pl.ds(ioff, _W)], ie[p], sems[p])

    def bfire(uscalar, iscalar, p):
        uoff = pl.multiple_of((uscalar >> 7) * _W, _W)
        ioff = pl.multiple_of((iscalar >> 7) * _W, _W)
        pltpu.async_copy(ubT_hbm.at[:, pl.ds(uoff, _W)], ub[p], bsems[p])
        pltpu.async_copy(ibT_hbm.at[:, pl.ds(ioff, _W)], ib[p], bsems[p])

    def drain(pu, p):
        pltpu.make_async_copy(
            uT_hbm.at[:, pl.ds(0, _W)], ue[pu], usems[pu]).wait()
        pltpu.make_async_copy(iT_hbm.at[:, pl.ds(0, _W)], ie[p], sems[p]).wait()

    def bdrain(p):
        pltpu.make_async_copy(
            ubT_hbm.at[:, pl.ds(0, _W)], ub[p], bsems[p]).wait()
        pltpu.make_async_copy(
            ibT_hbm.at[:, pl.ds(0, _W)], ib[p], bsems[p]).wait()

    u16p = uid_v[pl.ds(0, _L)]
    i16p = iid_v[pl.ds(0, _L)]
    for p in range(_RU):
        ufire(u16p[p], p)
    for p in range(_R):
        fire(i16p[p], p)
    for p in range(_RB):
        bfire(u16p[p], i16p[p], p)

    gb0 = gb_v[pl.ds(0, _L)][0]
    lane = lax.iota(jnp.int32, _L)
    zero16 = jnp.zeros((_L,), jnp.int32)

    def group(g, acc16):
        u16 = uid_v[pl.ds(g * _L, _L)]
        i16 = iid_v[pl.ds(g * _L, _L)]
        gnext = jnp.minimum(g + 1, _G - 1) * _L
        un16 = uid_v[pl.ds(gnext, _L)]
        in16 = iid_v[pl.ds(gnext, _L)]

        for j in range(_L):
            pu = j % _RU
            p = j % _R
            pb = j % _RB
            drain(pu, p)
            bdrain(pb)
            cu = jnp.full((_L,), u16[j] & (_W - 1), jnp.int32)
            ci = jnp.full((_L,), i16[j] & (_W - 1), jnp.int32)
            s = jnp.zeros((_L,), jnp.float32)
            for q in range(_D // _L):
                rows = q * _L + lane
                u = plsc.load_gather(ue[pu], [rows, cu])
                it = plsc.load_gather(ie[p], [rows, ci])
                s = s + u * it
            bias = (plsc.load_gather(ub[pb], [zero16, cu])
                    + plsc.load_gather(ib[pb], [zero16, ci]))

            if j < _L - _RU:
                ufire(u16[j + _RU], pu)
            else:
                @pl.when(g < _G - 1)
                def _(j=j, pu=pu):
                    ufire(un16[j + _RU - _L], pu)
            if j < _L - _R:
                fire(i16[j + _R], p)
            else:
                @pl.when(g < _G - 1)
                def _(j=j, p=p):
                    fire(in16[j + _R - _L], p)
            if j < _L - _RB:
                bfire(u16[j + _RB], i16[j + _RB], pb)
            else:
                @pl.when(g < _G - 1)
                def _(j=j, pb=pb):
                    bfire(un16[j + _RB - _L], in16[j + _RB - _L], pb)

            total = jnp.sum(s) + bias[0] + gb0
            acc16 = jnp.where(lane == j, total, acc16)

        out_v[pl.ds(g * _L, _L)] = acc16
        return acc16

    lax.fori_loop(0, _G, group, jnp.zeros((_L,), jnp.float32))

    pltpu.sync_copy(out_v, out_hbm.at[pl.ds(base, _BPW)])


def kernel(user_ids, item_ids, user_emb, item_emb, user_bias, item_bias,
           global_bias):
    uid = user_ids.astype(jnp.int32)
    iid = item_ids.astype(jnp.int32)
    return _mf_score(uid, iid, user_emb.T, item_emb.T, user_bias.T,
                     item_bias.T,
                     jnp.broadcast_to(global_bias.reshape(-1)[:1], (_L,)))
